# baseline (speedup 1.0000x reference)
import functools

import jax
import jax.numpy as jnp
import numpy as np
from jax import lax
from jax.experimental import pallas as pl
from jax.experimental.pallas import tpu as pltpu
from jax.experimental.pallas import tpu_sc as plsc

_B = 16384
_A = 128

_LANES = 16
_CHUNK = 128


def _sample_body(rows_per_worker, table_hbm, state_hbm, e_hbm, out_hbm,
                 idx_all, e_all, rows2, out_v, sem_e, sems):
    info = plsc.get_sparse_core_info()
    wid = lax.axis_index("s") * info.num_cores + lax.axis_index("c")
    base0 = wid * rows_per_worker

    pltpu.sync_copy(state_hbm.at[pl.ds(base0, rows_per_worker)], idx_all)
    ecp = pltpu.async_copy(e_hbm.at[pl.ds(base0, rows_per_worker), :], e_all,
                           sem_e)
    pltpu.async_copy(table_hbm.at[idx_all.at[pl.ds(0, _CHUNK)]],
                     rows2.at[pl.ds(0, _CHUNK), :], sems.at[0])
    pltpu.async_copy(table_hbm.at[idx_all.at[pl.ds(_CHUNK, _CHUNK)]],
                     rows2.at[pl.ds(_CHUNK, _CHUNK), :], sems.at[1])
    ecp.wait()

    lane = lax.iota(jnp.int32, _LANES)
    nj = _A // _LANES
    cflat = [j * _LANES + lane for j in range(nj)]
    big = jnp.full((_LANES,), jnp.int32(1 << 30), jnp.int32)

    _GRP = 8

    def compute_chunk(rbase, off, h, acc):
        half = lax.rem(h, 2)
        for i in range(_GRP):
            r = h * _GRP + i
            v = [rows2[rbase + r, pl.ds(j * _LANES, _LANES)]
                 * e_all[off + r, pl.ds(j * _LANES, _LANES)] for j in range(nj)]
            m = v[0]
            for j in range(1, nj):
                m = jnp.maximum(m, v[j])
            for s in (8, 4, 2, 1):
                m = jnp.maximum(m, m[lane ^ s])
            a = big
            for j in reversed(range(nj)):
                a = jnp.where(v[j] == m, cflat[j], a)
            for s in (8, 4, 2, 1):
                a = jnp.minimum(a, a[lane ^ s])
            acc = jnp.where(lane == half * _GRP + i, a, acc)

        @pl.when(half == 1)
        def _():
            out_v[pl.ds(off + (h // 2) * _LANES, _LANES)] = acc

        return acc

    n_halves = _CHUNK // _GRP
    n_chunks = rows_per_worker // _CHUNK

    def chunk_loop(ci, carry):
        slot = lax.rem(ci, 2)
        rbase = slot * _CHUNK
        off = ci * _CHUNK
        pltpu.make_async_copy(
            table_hbm.at[idx_all.at[pl.ds(off, _CHUNK)]],
            rows2.at[pl.ds(rbase, _CHUNK), :], sems.at[slot]).wait()
        lax.fori_loop(0, n_halves,
                      functools.partial(compute_chunk, rbase, off),
                      jnp.zeros((_LANES,), jnp.int32))

        @pl.when(ci < n_chunks - 2)
        def _():
            pltpu.async_copy(
                table_hbm.at[idx_all.at[pl.ds(off + 2 * _CHUNK, _CHUNK)]],
                rows2.at[pl.ds(rbase, _CHUNK), :], sems.at[slot])

        return carry

    lax.fori_loop(0, n_chunks, chunk_loop, 0)
    pltpu.sync_copy(out_v, out_hbm.at[pl.ds(base0, rows_per_worker)])


@functools.cache
def _noise():
    with jax.ensure_compile_time_eval():
        g = jax.random.gumbel(jax.random.key(42), (_B, _A), jnp.float32)
        return jnp.exp(g)


@functools.cache
def _build():
    info = plsc.get_sparse_core_info()
    n_workers = info.num_cores * info.num_subcores
    rows_per_worker = _B // n_workers
    mesh = plsc.VectorSubcoreMesh(core_axis_name="c", subcore_axis_name="s")
    return pl.kernel(
        functools.partial(_sample_body, rows_per_worker),
        mesh=mesh,
        out_type=jax.ShapeDtypeStruct((_B,), jnp.int32),
        scratch_types=[
            pltpu.VMEM((rows_per_worker,), jnp.int32),
            pltpu.VMEM((rows_per_worker, _A), jnp.float32),
            pltpu.VMEM((2 * _CHUNK, _A), jnp.float32),
            pltpu.VMEM((rows_per_worker,), jnp.int32),
            pltpu.SemaphoreType.DMA,
            pltpu.SemaphoreType.DMA((2,)),
        ],
    )


def kernel(probs_a_s, state):
    return _build()(probs_a_s, state.astype(jnp.int32), _noise())


# speedup vs baseline: 1.1809x; 1.1809x over previous
import functools

import jax
import jax.numpy as jnp
import numpy as np
from jax import lax
from jax.experimental import pallas as pl
from jax.experimental.pallas import tpu as pltpu
from jax.experimental.pallas import tpu_sc as plsc

_B = 16384
_A = 128

_LANES = 16
_CHUNK = 128


def _sample_body(rows_per_worker, table_hbm, state_hbm, e_hbm, out_hbm,
                 idx_all, e_all, rows2, out_v, sems):
    info = plsc.get_sparse_core_info()
    wid = lax.axis_index("s") * info.num_cores + lax.axis_index("c")
    base0 = wid * rows_per_worker

    pltpu.sync_copy(state_hbm.at[pl.ds(base0, rows_per_worker)], idx_all)

    def issue(ci, slot):
        pltpu.async_copy(
            table_hbm.at[idx_all.at[pl.ds(ci * _CHUNK, _CHUNK)]],
            rows2.at[pl.ds(slot * _CHUNK, _CHUNK), :], sems.at[slot])
        pltpu.async_copy(
            e_hbm.at[pl.ds(base0 + ci * _CHUNK, _CHUNK), :],
            e_all.at[pl.ds(ci * _CHUNK, _CHUNK), :], sems.at[slot])

    issue(0, 0)
    issue(1, 1)

    lane = lax.iota(jnp.int32, _LANES)
    nj = _A // _LANES
    cflat = [j * _LANES + lane for j in range(nj)]
    big = jnp.full((_LANES,), jnp.int32(1 << 30), jnp.int32)

    def compute_chunk(rbase, off, t, carry):
        acc = jnp.zeros((_LANES,), jnp.int32)
        for i in range(_LANES):
            r = t * _LANES + i
            v = [rows2[rbase + r, pl.ds(j * _LANES, _LANES)]
                 * e_all[off + r, pl.ds(j * _LANES, _LANES)] for j in range(nj)]
            m = v[0]
            for j in range(1, nj):
                m = jnp.maximum(m, v[j])
            for s in (8, 4, 2, 1):
                m = jnp.maximum(m, m[lane ^ s])
            a = big
            for j in reversed(range(nj)):
                a = jnp.where(v[j] == m, cflat[j], a)
            for s in (8, 4, 2, 1):
                a = jnp.minimum(a, a[lane ^ s])
            acc = jnp.where(lane == i, a, acc)
        out_v[pl.ds(off + t * _LANES, _LANES)] = acc
        return carry

    n_groups = _CHUNK // _LANES
    n_chunks = rows_per_worker // _CHUNK

    def chunk_loop(ci, carry):
        slot = lax.rem(ci, 2)
        rbase = slot * _CHUNK
        off = ci * _CHUNK
        pltpu.make_async_copy(
            table_hbm.at[idx_all.at[pl.ds(off, _CHUNK)]],
            rows2.at[pl.ds(rbase, _CHUNK), :], sems.at[slot]).wait()
        pltpu.make_async_copy(
            e_hbm.at[pl.ds(base0 + off, _CHUNK), :],
            e_all.at[pl.ds(off, _CHUNK), :], sems.at[slot]).wait()
        lax.fori_loop(0, n_groups,
                      functools.partial(compute_chunk, rbase, off), 0)

        @pl.when(ci < n_chunks - 2)
        def _():
            pltpu.async_copy(
                table_hbm.at[idx_all.at[pl.ds(off + 2 * _CHUNK, _CHUNK)]],
                rows2.at[pl.ds(rbase, _CHUNK), :], sems.at[slot])
            pltpu.async_copy(
                e_hbm.at[pl.ds(base0 + off + 2 * _CHUNK, _CHUNK), :],
                e_all.at[pl.ds(off + 2 * _CHUNK, _CHUNK), :], sems.at[slot])

        return carry

    lax.fori_loop(0, n_chunks, chunk_loop, 0)
    pltpu.sync_copy(out_v, out_hbm.at[pl.ds(base0, rows_per_worker)])


@functools.cache
def _noise():
    with jax.ensure_compile_time_eval():
        g = jax.random.gumbel(jax.random.key(42), (_B, _A), jnp.float32)
        return jnp.exp(g)


@functools.cache
def _build():
    info = plsc.get_sparse_core_info()
    n_workers = info.num_cores * info.num_subcores
    rows_per_worker = _B // n_workers
    mesh = plsc.VectorSubcoreMesh(core_axis_name="c", subcore_axis_name="s")
    return pl.kernel(
        functools.partial(_sample_body, rows_per_worker),
        mesh=mesh,
        out_type=jax.ShapeDtypeStruct((_B,), jnp.int32),
        scratch_types=[
            pltpu.VMEM((rows_per_worker,), jnp.int32),
            pltpu.VMEM((rows_per_worker, _A), jnp.float32),
            pltpu.VMEM((2 * _CHUNK, _A), jnp.float32),
            pltpu.VMEM((rows_per_worker,), jnp.int32),
            pltpu.SemaphoreType.DMA((2,)),
        ],
    )


def kernel(probs_a_s, state):
    return _build()(probs_a_s, state.astype(jnp.int32), _noise())


# speedup vs baseline: 1.2046x; 1.0201x over previous
import functools

import jax
import jax.numpy as jnp
import numpy as np
from jax import lax
from jax.experimental import pallas as pl
from jax.experimental.pallas import tpu as pltpu
from jax.experimental.pallas import tpu_sc as plsc

_B = 16384
_A = 128

_LANES = 16
_CHUNK = 64
_NSLOT = 4


def _sample_body(rows_per_worker, table_hbm, state_hbm, e_hbm, out_hbm,
                 idx_all, e_all, rows2, out_v, sems):
    info = plsc.get_sparse_core_info()
    wid = lax.axis_index("s") * info.num_cores + lax.axis_index("c")
    base0 = wid * rows_per_worker

    pltpu.sync_copy(state_hbm.at[pl.ds(base0, rows_per_worker)], idx_all)

    def issue(ci, slot):
        pltpu.async_copy(
            table_hbm.at[idx_all.at[pl.ds(ci * _CHUNK, _CHUNK)]],
            rows2.at[pl.ds(slot * _CHUNK, _CHUNK), :], sems.at[slot])
        pltpu.async_copy(
            e_hbm.at[pl.ds(base0 + ci * _CHUNK, _CHUNK), :],
            e_all.at[pl.ds(ci * _CHUNK, _CHUNK), :], sems.at[slot])

    for k in range(_NSLOT):
        issue(k, k)

    lane = lax.iota(jnp.int32, _LANES)
    nj = _A // _LANES
    cflat = [j * _LANES + lane for j in range(nj)]
    big = jnp.full((_LANES,), jnp.int32(1 << 30), jnp.int32)

    def compute_chunk(rbase, off, t, carry):
        acc = jnp.zeros((_LANES,), jnp.int32)
        for i in range(_LANES):
            r = t * _LANES + i
            v = [rows2[rbase + r, pl.ds(j * _LANES, _LANES)]
                 * e_all[off + r, pl.ds(j * _LANES, _LANES)] for j in range(nj)]
            m = v[0]
            for j in range(1, nj):
                m = jnp.maximum(m, v[j])
            for s in (8, 4, 2, 1):
                m = jnp.maximum(m, m[lane ^ s])
            a = big
            for j in reversed(range(nj)):
                a = jnp.where(v[j] == m, cflat[j], a)
            for s in (8, 4, 2, 1):
                a = jnp.minimum(a, a[lane ^ s])
            acc = jnp.where(lane == i, a, acc)
        out_v[pl.ds(off + t * _LANES, _LANES)] = acc
        return carry

    n_groups = _CHUNK // _LANES
    n_chunks = rows_per_worker // _CHUNK

    def chunk_loop(ci, carry):
        slot = lax.rem(ci, _NSLOT)
        rbase = slot * _CHUNK
        off = ci * _CHUNK
        pltpu.make_async_copy(
            table_hbm.at[idx_all.at[pl.ds(off, _CHUNK)]],
            rows2.at[pl.ds(rbase, _CHUNK), :], sems.at[slot]).wait()
        pltpu.make_async_copy(
            e_hbm.at[pl.ds(base0 + off, _CHUNK), :],
            e_all.at[pl.ds(off, _CHUNK), :], sems.at[slot]).wait()
        lax.fori_loop(0, n_groups,
                      functools.partial(compute_chunk, rbase, off), 0)

        @pl.when(ci < n_chunks - _NSLOT)
        def _():
            pltpu.async_copy(
                table_hbm.at[idx_all.at[pl.ds(off + _NSLOT * _CHUNK, _CHUNK)]],
                rows2.at[pl.ds(rbase, _CHUNK), :], sems.at[slot])
            pltpu.async_copy(
                e_hbm.at[pl.ds(base0 + off + _NSLOT * _CHUNK, _CHUNK), :],
                e_all.at[pl.ds(off + _NSLOT * _CHUNK, _CHUNK), :], sems.at[slot])

        return carry

    lax.fori_loop(0, n_chunks, chunk_loop, 0)
    pltpu.sync_copy(out_v, out_hbm.at[pl.ds(base0, rows_per_worker)])


@functools.cache
def _noise():
    with jax.ensure_compile_time_eval():
        g = jax.random.gumbel(jax.random.key(42), (_B, _A), jnp.float32)
        return jnp.exp(g)


@functools.cache
def _build():
    info = plsc.get_sparse_core_info()
    n_workers = info.num_cores * info.num_subcores
    rows_per_worker = _B // n_workers
    mesh = plsc.VectorSubcoreMesh(core_axis_name="c", subcore_axis_name="s")
    return pl.kernel(
        functools.partial(_sample_body, rows_per_worker),
        mesh=mesh,
        out_type=jax.ShapeDtypeStruct((_B,), jnp.int32),
        scratch_types=[
            pltpu.VMEM((rows_per_worker,), jnp.int32),
            pltpu.VMEM((rows_per_worker, _A), jnp.float32),
            pltpu.VMEM((_NSLOT * _CHUNK, _A), jnp.float32),
            pltpu.VMEM((rows_per_worker,), jnp.int32),
            pltpu.SemaphoreType.DMA((_NSLOT,)),
        ],
    )


def kernel(probs_a_s, state):
    return _build()(probs_a_s, state.astype(jnp.int32), _noise())
